# baseline (speedup 1.0000x reference)
import functools

import jax
import jax.numpy as jnp
from jax import lax
from jax.experimental import pallas as pl
from jax.experimental.pallas import tpu as pltpu
from jax.experimental.pallas import tpu_sc as plsc

N = 10000
E = 320000
D = 128
LANES = 16
NC = 2
NS = 16
NW = NC * NS
EPT = E // NW
CHUNK = 40
NCHUNK = EPT // CHUNK
NBUF = 5
PREF = 3
NROUND = NCHUNK // NBUF
TAIL = NCHUNK - NROUND * NBUF
GRP = 8
RPT = 632
RPT_LAST = N - (NS - 1) * RPT


def _sc_aggregate(x, col1d, row1d, w1d):
    mesh = plsc.VectorSubcoreMesh(core_axis_name="c", subcore_axis_name="s")

    @functools.partial(
        pl.kernel,
        mesh=mesh,
        out_type=jax.ShapeDtypeStruct((NC * N, D), jnp.float32),
        scratch_types=(
            [
                pltpu.VMEM_SHARED((N, D), jnp.float32),
                pltpu.VMEM((EPT,), jnp.int32),
                pltpu.VMEM((EPT + LANES,), jnp.float32),
            ]
            + [pltpu.VMEM((CHUNK, D), jnp.float32)] * NBUF
            + [pltpu.VMEM((CHUNK,), jnp.int32)] * NBUF
            + [pltpu.SemaphoreType.DMA] * (3 * NBUF)
        ),
    )
    def k(x_hbm, col_hbm, row_hbm, w_hbm, out_hbm,
          acc, col_v, w_v, *ring):
        bufs = ring[0:NBUF]
        ridxs = ring[NBUF:2 * NBUF]
        gsems = ring[2 * NBUF:3 * NBUF]
        ssems = ring[3 * NBUF:4 * NBUF]
        risems = ring[4 * NBUF:5 * NBUF]
        buf0 = bufs[0]
        cid = lax.axis_index("c")
        sid = lax.axis_index("s")
        wid = cid * NS + sid

        pltpu.sync_copy(col_hbm.at[pl.ds(wid * EPT, EPT)], col_v)
        pltpu.sync_copy(w_hbm.at[pl.ds(wid * EPT, EPT)], w_v.at[pl.ds(0, EPT)])

        zeros = jnp.zeros((LANES,), jnp.float32)

        def zbody(j, c_):
            for c in range(D // LANES):
                buf0[j, pl.ds(c * LANES, LANES)] = zeros
            return c_

        lax.fori_loop(0, CHUNK, zbody, 0)
        r0 = sid * RPT

        def zero_rows(base, nrows):
            for i in range(nrows // CHUNK):
                pltpu.sync_copy(buf0, acc.at[pl.ds(base + i * CHUNK, CHUNK)])
            rem = nrows % CHUNK
            if rem:
                pltpu.sync_copy(buf0.at[pl.ds(0, rem)],
                                acc.at[pl.ds(base + (nrows // CHUNK) * CHUNK,
                                             rem)])

        zero_rows(r0, RPT_LAST)

        @pl.when(sid < NS - 1)
        def _():
            zero_rows(r0 + RPT_LAST, RPT - RPT_LAST)

        plsc.subcore_barrier()

        def gather_start(g, buf, sem):
            pltpu.make_async_copy(
                x_hbm.at[col_v.at[pl.ds(g * CHUNK, CHUNK)]], buf, sem).start()

        def gather_wait(g, buf, sem):
            pltpu.make_async_copy(
                x_hbm.at[col_v.at[pl.ds(g * CHUNK, CHUNK)]], buf, sem).wait()

        def ridx_start(g, ridx, sem):
            pltpu.make_async_copy(
                row_hbm.at[pl.ds(wid * EPT + g * CHUNK, CHUNK)], ridx,
                sem).start()

        def ridx_wait(g, ridx, sem):
            pltpu.make_async_copy(
                row_hbm.at[pl.ds(wid * EPT + g * CHUNK, CHUNK)], ridx,
                sem).wait()

        def do_chunk(g, buf, ridx, ssem, risem):
            wbase = g * CHUNK

            def grp_body(jj, c_):
                base = wbase + jj * GRP
                w16 = w_v[pl.ds(base, LANES)]
                j0 = jj * GRP
                for l in range(GRP):
                    wsplat = jnp.broadcast_to(w16[l], (LANES,))
                    j = j0 + l
                    for c in range(D // LANES):
                        sl = pl.ds(c * LANES, LANES)
                        buf[j, sl] = buf[j, sl] * wsplat
                return c_

            lax.fori_loop(0, CHUNK // GRP, grp_body, 0)
            ridx_wait(g, ridx, risem)
            pltpu.async_copy(buf, acc.at[ridx], ssem, add=True)

        def scatter_wait(buf, ridx, ssem):
            pltpu.make_async_copy(buf, acc.at[ridx], ssem).wait()

        for b in range(PREF):
            ridx_start(b, ridxs[b], risems[b])
            gather_start(b, bufs[b], gsems[b])

        def round_body(q, c_):
            for b in range(NBUF):
                g = q * NBUF + b
                gather_wait(g, bufs[b], gsems[b])
                do_chunk(g, bufs[b], ridxs[b], ssems[b], risems[b])
                bb = (b + PREF) % NBUF

                @pl.when(g >= NBUF - PREF)
                def _(bb=bb):
                    scatter_wait(bufs[bb], ridxs[bb], ssems[bb])

                @pl.when(g + PREF < NCHUNK)
                def _(bb=bb):
                    ridx_start(g + PREF, ridxs[bb], risems[bb])
                    gather_start(g + PREF, bufs[bb], gsems[bb])

            return c_

        lax.fori_loop(0, NROUND, round_body, 0)

        for gl in range(NCHUNK - (NBUF - PREF), NCHUNK):
            b = gl % NBUF
            scatter_wait(bufs[b], ridxs[b], ssems[b])

        plsc.subcore_barrier()

        @pl.when(sid < NS - 1)
        def _():
            pltpu.sync_copy(acc.at[pl.ds(r0, RPT)],
                            out_hbm.at[pl.ds(cid * N + r0, RPT)])

        @pl.when(sid == NS - 1)
        def _():
            pltpu.sync_copy(acc.at[pl.ds(r0, RPT_LAST)],
                            out_hbm.at[pl.ds(cid * N + r0, RPT_LAST)])

    return k(x, col1d, row1d, w1d)


def _tc_body(x_ref, p0_ref, p1_ref, ws_ref, wn_ref, b_ref, o_ref):
    dn = (((1,), (1,)), ((), ()))
    o_ref[...] = (
        lax.dot_general(x_ref[...], ws_ref[...], dn,
                        preferred_element_type=jnp.float32)
        + b_ref[...]
        + lax.dot_general(p0_ref[...] + p1_ref[...], wn_ref[...], dn,
                          preferred_element_type=jnp.float32)
    )


def _tc_combine(x, partial, W_self, W_neigh, b2d):
    BM = 1000
    nblk = N // BM
    return pl.pallas_call(
        _tc_body,
        grid=(nblk,),
        in_specs=[
            pl.BlockSpec((BM, D), lambda i: (i, 0)),
            pl.BlockSpec((BM, D), lambda i: (i, 0)),
            pl.BlockSpec((BM, D), lambda i, _n=nblk: (i + _n, 0)),
            pl.BlockSpec((D, D), lambda i: (0, 0)),
            pl.BlockSpec((D, D), lambda i: (0, 0)),
            pl.BlockSpec((1, D), lambda i: (0, 0)),
        ],
        out_specs=pl.BlockSpec((BM, D), lambda i: (i, 0)),
        out_shape=jax.ShapeDtypeStruct((N, D), jnp.float32),
    )(x, partial, partial, W_self, W_neigh, b2d)


def kernel(x, edge_index, edge_weight, W_self, b_self, W_neigh):
    row1d = edge_index[0].astype(jnp.int32)
    col1d = edge_index[1].astype(jnp.int32)
    w1d = edge_weight.astype(jnp.float32)
    partial = _sc_aggregate(x, col1d, row1d, w1d)
    return _tc_combine(x, partial, W_self, W_neigh, b_self.reshape(1, D))


# speedup vs baseline: 1.8893x; 1.8893x over previous
import functools

import jax
import jax.numpy as jnp
from jax import lax
from jax.experimental import pallas as pl
from jax.experimental.pallas import tpu as pltpu
from jax.experimental.pallas import tpu_sc as plsc

N = 10000
E = 320000
D = 128
LANES = 16
NC = 2
NS = 16
NW = NC * NS
EPT = E // NW
CHUNK = 40
NCHUNK = EPT // CHUNK
NBUF = 5
PREF = 3
NROUND = NCHUNK // NBUF
TAIL = NCHUNK - NROUND * NBUF
GRP = 8
RPT = 632
RPT_LAST = N - (NS - 1) * RPT


def _sc_aggregate(x, col1d, row1d, w1d):
    mesh = plsc.VectorSubcoreMesh(core_axis_name="c", subcore_axis_name="s")

    @functools.partial(
        pl.kernel,
        mesh=mesh,
        out_type=jax.ShapeDtypeStruct((NC * N, D), jnp.float32),
        scratch_types=(
            [
                pltpu.VMEM_SHARED((N, D), jnp.float32),
                pltpu.VMEM((EPT,), jnp.int32),
                pltpu.VMEM((EPT + LANES,), jnp.float32),
            ]
            + [pltpu.VMEM((CHUNK, D), jnp.float32)] * NBUF
            + [pltpu.VMEM((CHUNK,), jnp.int32)] * NBUF
            + [pltpu.SemaphoreType.DMA] * (3 * NBUF)
        ),
    )
    def k(x_hbm, col_hbm, row_hbm, w_hbm, out_hbm,
          acc, col_v, w_v, *ring):
        bufs = ring[0:NBUF]
        ridxs = ring[NBUF:2 * NBUF]
        gsems = ring[2 * NBUF:3 * NBUF]
        ssems = ring[3 * NBUF:4 * NBUF]
        risems = ring[4 * NBUF:5 * NBUF]
        buf0 = bufs[0]
        cid = lax.axis_index("c")
        sid = lax.axis_index("s")
        wid = cid * NS + sid

        pltpu.sync_copy(col_hbm.at[pl.ds(wid * EPT, EPT)], col_v)
        pltpu.sync_copy(w_hbm.at[pl.ds(wid * EPT, EPT)], w_v.at[pl.ds(0, EPT)])

        zeros = jnp.zeros((LANES,), jnp.float32)

        def zbody(j, c_):
            for c in range(D // LANES):
                buf0[j, pl.ds(c * LANES, LANES)] = zeros
            return c_

        lax.fori_loop(0, CHUNK, zbody, 0)
        r0 = sid * RPT

        def zero_rows(base, nrows):
            for i in range(nrows // CHUNK):
                pltpu.sync_copy(buf0, acc.at[pl.ds(base + i * CHUNK, CHUNK)])
            rem = nrows % CHUNK
            if rem:
                pltpu.sync_copy(buf0.at[pl.ds(0, rem)],
                                acc.at[pl.ds(base + (nrows // CHUNK) * CHUNK,
                                             rem)])

        zero_rows(r0, RPT_LAST)

        @pl.when(sid < NS - 1)
        def _():
            zero_rows(r0 + RPT_LAST, RPT - RPT_LAST)

        plsc.subcore_barrier()

        def gather_start(g, buf, sem):
            pltpu.make_async_copy(
                x_hbm.at[col_v.at[pl.ds(g * CHUNK, CHUNK)]], buf, sem).start()

        def gather_wait(g, buf, sem):
            pltpu.make_async_copy(
                x_hbm.at[col_v.at[pl.ds(g * CHUNK, CHUNK)]], buf, sem).wait()

        def ridx_start(g, ridx, sem):
            pltpu.make_async_copy(
                row_hbm.at[pl.ds(wid * EPT + g * CHUNK, CHUNK)], ridx,
                sem).start()

        def ridx_wait(g, ridx, sem):
            pltpu.make_async_copy(
                row_hbm.at[pl.ds(wid * EPT + g * CHUNK, CHUNK)], ridx,
                sem).wait()

        def do_chunk(g, buf, ridx, ssem, risem):
            wbase = g * CHUNK
            for jj in range(CHUNK // LANES):
                w16 = w_v[pl.ds(wbase + jj * LANES, LANES)]
                for l in range(LANES):
                    j = jj * LANES + l
                    wsplat = jnp.broadcast_to(w16[l], (LANES,))
                    for c in range(D // LANES):
                        sl = pl.ds(c * LANES, LANES)
                        buf[j, sl] = buf[j, sl] * wsplat
            if CHUNK % LANES:
                wtail = w_v[pl.ds(wbase + CHUNK - LANES, LANES)]
                for j in range((CHUNK // LANES) * LANES, CHUNK):
                    l = j - (CHUNK - LANES)
                    wsplat = jnp.broadcast_to(wtail[l], (LANES,))
                    for c in range(D // LANES):
                        sl = pl.ds(c * LANES, LANES)
                        buf[j, sl] = buf[j, sl] * wsplat
            ridx_wait(g, ridx, risem)
            pltpu.async_copy(buf, acc.at[ridx], ssem, add=True)

        def scatter_wait(buf, ridx, ssem):
            pltpu.make_async_copy(buf, acc.at[ridx], ssem).wait()

        for b in range(PREF):
            ridx_start(b, ridxs[b], risems[b])
            gather_start(b, bufs[b], gsems[b])

        def round_body(q, c_):
            for b in range(NBUF):
                g = q * NBUF + b
                gather_wait(g, bufs[b], gsems[b])
                do_chunk(g, bufs[b], ridxs[b], ssems[b], risems[b])
                bb = (b + PREF) % NBUF

                @pl.when(g >= NBUF - PREF)
                def _(bb=bb):
                    scatter_wait(bufs[bb], ridxs[bb], ssems[bb])

                @pl.when(g + PREF < NCHUNK)
                def _(bb=bb):
                    ridx_start(g + PREF, ridxs[bb], risems[bb])
                    gather_start(g + PREF, bufs[bb], gsems[bb])

            return c_

        lax.fori_loop(0, NROUND, round_body, 0)

        for gl in range(NCHUNK - (NBUF - PREF), NCHUNK):
            b = gl % NBUF
            scatter_wait(bufs[b], ridxs[b], ssems[b])

        plsc.subcore_barrier()

        @pl.when(sid < NS - 1)
        def _():
            pltpu.sync_copy(acc.at[pl.ds(r0, RPT)],
                            out_hbm.at[pl.ds(cid * N + r0, RPT)])

        @pl.when(sid == NS - 1)
        def _():
            pltpu.sync_copy(acc.at[pl.ds(r0, RPT_LAST)],
                            out_hbm.at[pl.ds(cid * N + r0, RPT_LAST)])

    return k(x, col1d, row1d, w1d)


def _tc_body(x_ref, p0_ref, p1_ref, ws_ref, wn_ref, b_ref, o_ref):
    dn = (((1,), (1,)), ((), ()))
    o_ref[...] = (
        lax.dot_general(x_ref[...], ws_ref[...], dn,
                        preferred_element_type=jnp.float32)
        + b_ref[...]
        + lax.dot_general(p0_ref[...] + p1_ref[...], wn_ref[...], dn,
                          preferred_element_type=jnp.float32)
    )


def _tc_combine(x, partial, W_self, W_neigh, b2d):
    BM = 1000
    nblk = N // BM
    return pl.pallas_call(
        _tc_body,
        grid=(nblk,),
        in_specs=[
            pl.BlockSpec((BM, D), lambda i: (i, 0)),
            pl.BlockSpec((BM, D), lambda i: (i, 0)),
            pl.BlockSpec((BM, D), lambda i, _n=nblk: (i + _n, 0)),
            pl.BlockSpec((D, D), lambda i: (0, 0)),
            pl.BlockSpec((D, D), lambda i: (0, 0)),
            pl.BlockSpec((1, D), lambda i: (0, 0)),
        ],
        out_specs=pl.BlockSpec((BM, D), lambda i: (i, 0)),
        out_shape=jax.ShapeDtypeStruct((N, D), jnp.float32),
    )(x, partial, partial, W_self, W_neigh, b2d)


def kernel(x, edge_index, edge_weight, W_self, b_self, W_neigh):
    row1d = edge_index[0].astype(jnp.int32)
    col1d = edge_index[1].astype(jnp.int32)
    w1d = edge_weight.astype(jnp.float32)
    partial = _sc_aggregate(x, col1d, row1d, w1d)
    return _tc_combine(x, partial, W_self, W_neigh, b_self.reshape(1, D))
